# baseline (speedup 1.0000x reference)
import functools

import jax
import jax.numpy as jnp
from jax import lax
from jax.experimental import pallas as pl
from jax.experimental.pallas import tpu as pltpu
from jax.experimental.pallas import tpu_sc as plsc

H = 1024
BZ = 1024
KB = 4096
STEPS = 4
KT = 1024
NK = STEPS * KB
D = (512, 256, 256)
OFF = (0, 512, 768)

_IDX_BITS = 10
_IDX_MASK = (1 << _IDX_BITS) - 1
_ONE_BITS = 0x3F800000

_NC, _NS = 2, 16
_NW = _NC * _NS
_B_PER_W = BZ // _NW


def _search_body(query_ref, keys_ref, w_ref, vbits_ref, idx_ref, qp_ref,
                 kp0_ref, kp1_ref, kp2_ref, qnt_ref, wbf_ref):
    i = pl.program_id(0)
    j = pl.program_id(1)

    @pl.when((i == 0) & (j == 0))
    def _init():
        w = w_ref[...]
        wbf = w.astype(jnp.bfloat16)
        wbf_ref[...] = wbf
        qp = lax.dot(query_ref[...].astype(jnp.bfloat16), wbf,
                     preferred_element_type=jnp.float32)
        qp_ref[...] = qp
        for u in range(3):
            q = qp[:, OFF[u]:OFF[u] + D[u]]
            n = jnp.sqrt(jnp.sum(q * q, axis=1, keepdims=True))
            qn = q / jnp.maximum(n, 1e-8)
            qnt_ref[OFF[u]:OFF[u] + D[u], :] = qn.T.astype(jnp.bfloat16)
        vbits_ref[...] = jnp.full((8, BZ), _ONE_BITS, jnp.int32)
        idx_ref[...] = jnp.zeros((8, BZ), jnp.int32)

    keys_bf = keys_ref[0].astype(jnp.bfloat16)
    kp = lax.dot(keys_bf, wbf_ref[...],
                 preferred_element_type=jnp.float32)
    kp_bf = kp.astype(jnp.bfloat16)
    base = i * KB + j * KT
    row = lax.broadcasted_iota(jnp.int32, (KT, BZ), 0)
    rbits = _IDX_MASK - row
    for u, kpu_ref in ((0, kp0_ref), (1, kp1_ref), (2, kp2_ref)):
        kpu = kp[:, OFF[u]:OFF[u] + D[u]]
        kpu_ref[...] = kpu
        inv = 1.0 / jnp.maximum(
            jnp.sqrt(jnp.sum(kpu * kpu, axis=1, keepdims=True)), 1e-3)
        st = lax.dot(kp_bf[:, OFF[u]:OFF[u] + D[u]],
                     qnt_ref[OFF[u]:OFF[u] + D[u], :],
                     preferred_element_type=jnp.float32)
        packed = (lax.bitcast_convert_type(st * inv + 2.0, jnp.int32)
                  & jnp.int32(~_IDX_MASK)) | rbits
        pmax = jnp.max(packed, axis=0)
        pv = pmax & jnp.int32(~_IDX_MASK)
        gidx = base + (_IDX_MASK - (pmax & _IDX_MASK))
        better = pv > vbits_ref[u, :]
        vbits_ref[u, :] = jnp.where(better, pv, vbits_ref[u, :])
        idx_ref[u, :] = jnp.where(better, gidx, idx_ref[u, :])


_search = pl.pallas_call(
    _search_body,
    grid=(STEPS, KB // KT),
    in_specs=[
        pl.BlockSpec((BZ, H), lambda i, j: (0, 0)),
        pl.BlockSpec((1, KT, H), lambda i, j: (i, j, 0)),
        pl.BlockSpec((H, H), lambda i, j: (0, 0)),
    ],
    out_specs=[
        pl.BlockSpec((8, BZ), lambda i, j: (0, 0)),
        pl.BlockSpec((8, BZ), lambda i, j: (0, 0)),
        pl.BlockSpec((BZ, H), lambda i, j: (0, 0)),
        pl.BlockSpec((KT, D[0]), lambda i, j: (i * (KB // KT) + j, 0)),
        pl.BlockSpec((KT, D[1]), lambda i, j: (i * (KB // KT) + j, 0)),
        pl.BlockSpec((KT, D[2]), lambda i, j: (i * (KB // KT) + j, 0)),
    ],
    out_shape=[
        jax.ShapeDtypeStruct((8, BZ), jnp.int32),
        jax.ShapeDtypeStruct((8, BZ), jnp.int32),
        jax.ShapeDtypeStruct((BZ, H), jnp.float32),
        jax.ShapeDtypeStruct((NK, D[0]), jnp.float32),
        jax.ShapeDtypeStruct((NK, D[1]), jnp.float32),
        jax.ShapeDtypeStruct((NK, D[2]), jnp.float32),
    ],
    scratch_shapes=[
        pltpu.VMEM((H, BZ), jnp.bfloat16),
        pltpu.VMEM((H, H), jnp.bfloat16),
    ],
)


def _gather_diag(kp0, kp1, kp2, idx0, idx1, idx2):
    mesh = plsc.VectorSubcoreMesh(core_axis_name="c", subcore_axis_name="s")

    @functools.partial(
        pl.kernel, mesh=mesh,
        out_type=[
            jax.ShapeDtypeStruct((BZ, D[0]), jnp.float32),
            jax.ShapeDtypeStruct((BZ, D[1]), jnp.float32),
            jax.ShapeDtypeStruct((BZ, D[2]), jnp.float32),
        ],
        scratch_types=[
            pltpu.VMEM((_B_PER_W,), jnp.int32),
            pltpu.VMEM((_B_PER_W, D[0]), jnp.float32),
            pltpu.VMEM((_B_PER_W, D[1]), jnp.float32),
            pltpu.VMEM((_B_PER_W, D[2]), jnp.float32),
            pltpu.SemaphoreType.DMA,
        ],
    )
    def k(kp0_hbm, kp1_hbm, kp2_hbm, i0_hbm, i1_hbm, i2_hbm,
          o0_hbm, o1_hbm, o2_hbm, idx_v, r0_v, r1_v, r2_v, sem):
        wid = lax.axis_index("s") * _NC + lax.axis_index("c")
        base = wid * _B_PER_W
        for i_hbm, kp_hbm, r_v, o_hbm in (
                (i0_hbm, kp0_hbm, r0_v, o0_hbm),
                (i1_hbm, kp1_hbm, r1_v, o1_hbm),
                (i2_hbm, kp2_hbm, r2_v, o2_hbm)):
            pltpu.sync_copy(i_hbm.at[pl.ds(base, _B_PER_W)], idx_v)
            pltpu.async_copy(kp_hbm.at[idx_v], r_v, sem).wait()
            pltpu.sync_copy(r_v, o_hbm.at[pl.ds(base, _B_PER_W)])

    return k(kp0, kp1, kp2, idx0, idx1, idx2)


def _final_body(qp_ref, d0_ref, d1_ref, d2_ref, vbits_ref, out_ref):
    acc = jnp.float32(0.0)
    for u, d_ref in ((0, d0_ref), (1, d1_ref), (2, d2_ref)):
        diag = d_ref[...]
        alive = vbits_ref[u, :] > jnp.int32(_ONE_BITS)
        q = qp_ref[:, OFF[u]:OFF[u] + D[u]]
        dot = jnp.sum(q * diag, axis=1)
        nq = jnp.sqrt(jnp.sum(q * q, axis=1))
        nd = jnp.sqrt(jnp.sum(diag * diag, axis=1))
        cos = jnp.where(alive, dot / jnp.maximum(nq * nd, 1e-8), 0.0)
        acc = acc + D[u] * jnp.sum(cos)
    loss = -acc / (BZ * H)
    out_ref[...] = jnp.full((8, 128), loss, jnp.float32)


_final = pl.pallas_call(
    _final_body,
    in_specs=[
        pl.BlockSpec((BZ, H), lambda: (0, 0)),
        pl.BlockSpec((BZ, D[0]), lambda: (0, 0)),
        pl.BlockSpec((BZ, D[1]), lambda: (0, 0)),
        pl.BlockSpec((BZ, D[2]), lambda: (0, 0)),
        pl.BlockSpec((8, BZ), lambda: (0, 0)),
    ],
    out_specs=pl.BlockSpec((8, 128), lambda: (0, 0)),
    out_shape=jax.ShapeDtypeStruct((8, 128), jnp.float32),
)


def kernel(query, keys, W):
    vbits, idx, qp, kp0, kp1, kp2 = _search(query, keys, W)
    d0, d1, d2 = _gather_diag(kp0, kp1, kp2, idx[0], idx[1], idx[2])
    out = _final(qp, d0, d1, d2, vbits)
    return out[0, 0]


# speedup vs baseline: 1.0536x; 1.0536x over previous
import functools

import jax
import jax.numpy as jnp
from jax import lax
from jax.experimental import pallas as pl
from jax.experimental.pallas import tpu as pltpu
from jax.experimental.pallas import tpu_sc as plsc

H = 1024
BZ = 1024
KB = 4096
STEPS = 4
KT = 2048
NK = STEPS * KB
D = (512, 256, 256)
OFF = (0, 512, 768)

_IDX_BITS = (KT - 1).bit_length()
_IDX_MASK = (1 << _IDX_BITS) - 1
_ONE_BITS = 0x3F800000

_NC, _NS = 2, 16
_NW = _NC * _NS
_B_PER_W = BZ // _NW


def _search_body(query_ref, keys_ref, w_ref, vbits_ref, idx_ref, qp_ref,
                 kp0_ref, kp1_ref, kp2_ref, qnt_ref, wbf_ref):
    i = pl.program_id(0)
    j = pl.program_id(1)

    @pl.when((i == 0) & (j == 0))
    def _init():
        w = w_ref[...]
        wbf = w.astype(jnp.bfloat16)
        wbf_ref[...] = wbf
        qp = lax.dot(query_ref[...].astype(jnp.bfloat16), wbf,
                     preferred_element_type=jnp.float32)
        qp_ref[...] = qp
        for u in range(3):
            q = qp[:, OFF[u]:OFF[u] + D[u]]
            n = jnp.sqrt(jnp.sum(q * q, axis=1, keepdims=True))
            qn = q / jnp.maximum(n, 1e-8)
            qnt_ref[OFF[u]:OFF[u] + D[u], :] = qn.T.astype(jnp.bfloat16)
        vbits_ref[...] = jnp.full((8, BZ), _ONE_BITS, jnp.int32)
        idx_ref[...] = jnp.zeros((8, BZ), jnp.int32)

    keys_bf = keys_ref[0].astype(jnp.bfloat16)
    kp = lax.dot(keys_bf, wbf_ref[...],
                 preferred_element_type=jnp.float32)
    base = i * KB + j * KT
    row = lax.broadcasted_iota(jnp.int32, (KT, BZ), 0)
    rbits = _IDX_MASK - row
    for u, kpu_ref in ((0, kp0_ref), (1, kp1_ref), (2, kp2_ref)):
        kpu = kp[:, OFF[u]:OFF[u] + D[u]]
        kpu_ref[...] = kpu
        inv = 1.0 / jnp.maximum(
            jnp.sqrt(jnp.sum(kpu * kpu, axis=1, keepdims=True)), 1e-3)
        kpn = (kpu * inv).astype(jnp.bfloat16)
        st = lax.dot(kpn, qnt_ref[OFF[u]:OFF[u] + D[u], :],
                     preferred_element_type=jnp.float32)
        packed = (lax.bitcast_convert_type(st + 2.0, jnp.int32)
                  & jnp.int32(~_IDX_MASK)) | rbits
        pmax = jnp.max(packed, axis=0)
        pv = pmax & jnp.int32(~_IDX_MASK)
        gidx = base + (_IDX_MASK - (pmax & _IDX_MASK))
        better = pv > vbits_ref[u, :]
        vbits_ref[u, :] = jnp.where(better, pv, vbits_ref[u, :])
        idx_ref[u, :] = jnp.where(better, gidx, idx_ref[u, :])


_search = pl.pallas_call(
    _search_body,
    grid=(STEPS, KB // KT),
    in_specs=[
        pl.BlockSpec((BZ, H), lambda i, j: (0, 0)),
        pl.BlockSpec((1, KT, H), lambda i, j: (i, j, 0)),
        pl.BlockSpec((H, H), lambda i, j: (0, 0)),
    ],
    out_specs=[
        pl.BlockSpec((8, BZ), lambda i, j: (0, 0)),
        pl.BlockSpec((8, BZ), lambda i, j: (0, 0)),
        pl.BlockSpec((BZ, H), lambda i, j: (0, 0)),
        pl.BlockSpec((KT, D[0]), lambda i, j: (i * (KB // KT) + j, 0)),
        pl.BlockSpec((KT, D[1]), lambda i, j: (i * (KB // KT) + j, 0)),
        pl.BlockSpec((KT, D[2]), lambda i, j: (i * (KB // KT) + j, 0)),
    ],
    out_shape=[
        jax.ShapeDtypeStruct((8, BZ), jnp.int32),
        jax.ShapeDtypeStruct((8, BZ), jnp.int32),
        jax.ShapeDtypeStruct((BZ, H), jnp.float32),
        jax.ShapeDtypeStruct((NK, D[0]), jnp.float32),
        jax.ShapeDtypeStruct((NK, D[1]), jnp.float32),
        jax.ShapeDtypeStruct((NK, D[2]), jnp.float32),
    ],
    scratch_shapes=[
        pltpu.VMEM((H, BZ), jnp.bfloat16),
        pltpu.VMEM((H, H), jnp.bfloat16),
    ],
)


def _gather_diag(kp0, kp1, kp2, idx0, idx1, idx2):
    mesh = plsc.VectorSubcoreMesh(core_axis_name="c", subcore_axis_name="s")

    @functools.partial(
        pl.kernel, mesh=mesh,
        out_type=[
            jax.ShapeDtypeStruct((BZ, D[0]), jnp.float32),
            jax.ShapeDtypeStruct((BZ, D[1]), jnp.float32),
            jax.ShapeDtypeStruct((BZ, D[2]), jnp.float32),
        ],
        scratch_types=[
            pltpu.VMEM((_B_PER_W,), jnp.int32),
            pltpu.VMEM((_B_PER_W, D[0]), jnp.float32),
            pltpu.VMEM((_B_PER_W, D[1]), jnp.float32),
            pltpu.VMEM((_B_PER_W, D[2]), jnp.float32),
            pltpu.SemaphoreType.DMA,
        ],
    )
    def k(kp0_hbm, kp1_hbm, kp2_hbm, i0_hbm, i1_hbm, i2_hbm,
          o0_hbm, o1_hbm, o2_hbm, idx_v, r0_v, r1_v, r2_v, sem):
        wid = lax.axis_index("s") * _NC + lax.axis_index("c")
        base = wid * _B_PER_W
        for i_hbm, kp_hbm, r_v, o_hbm in (
                (i0_hbm, kp0_hbm, r0_v, o0_hbm),
                (i1_hbm, kp1_hbm, r1_v, o1_hbm),
                (i2_hbm, kp2_hbm, r2_v, o2_hbm)):
            pltpu.sync_copy(i_hbm.at[pl.ds(base, _B_PER_W)], idx_v)
            pltpu.async_copy(kp_hbm.at[idx_v], r_v, sem).wait()
            pltpu.sync_copy(r_v, o_hbm.at[pl.ds(base, _B_PER_W)])

    return k(kp0, kp1, kp2, idx0, idx1, idx2)


def _final_body(qp_ref, d0_ref, d1_ref, d2_ref, vbits_ref, out_ref):
    acc = jnp.float32(0.0)
    for u, d_ref in ((0, d0_ref), (1, d1_ref), (2, d2_ref)):
        diag = d_ref[...]
        alive = vbits_ref[u, :] > jnp.int32(_ONE_BITS)
        q = qp_ref[:, OFF[u]:OFF[u] + D[u]]
        dot = jnp.sum(q * diag, axis=1)
        nq = jnp.sqrt(jnp.sum(q * q, axis=1))
        nd = jnp.sqrt(jnp.sum(diag * diag, axis=1))
        cos = jnp.where(alive, dot / jnp.maximum(nq * nd, 1e-8), 0.0)
        acc = acc + D[u] * jnp.sum(cos)
    loss = -acc / (BZ * H)
    out_ref[...] = jnp.full((8, 128), loss, jnp.float32)


_final = pl.pallas_call(
    _final_body,
    in_specs=[
        pl.BlockSpec((BZ, H), lambda: (0, 0)),
        pl.BlockSpec((BZ, D[0]), lambda: (0, 0)),
        pl.BlockSpec((BZ, D[1]), lambda: (0, 0)),
        pl.BlockSpec((BZ, D[2]), lambda: (0, 0)),
        pl.BlockSpec((8, BZ), lambda: (0, 0)),
    ],
    out_specs=pl.BlockSpec((8, 128), lambda: (0, 0)),
    out_shape=jax.ShapeDtypeStruct((8, 128), jnp.float32),
)


def kernel(query, keys, W):
    vbits, idx, qp, kp0, kp1, kp2 = _search(query, keys, W)
    d0, d1, d2 = _gather_diag(kp0, kp1, kp2, idx[0], idx[1], idx[2])
    out = _final(qp, d0, d1, d2, vbits)
    return out[0, 0]


# speedup vs baseline: 1.0834x; 1.0283x over previous
import functools

import jax
import jax.numpy as jnp
from jax import lax
from jax.experimental import pallas as pl
from jax.experimental.pallas import tpu as pltpu
from jax.experimental.pallas import tpu_sc as plsc

H = 1024
BZ = 1024
KB = 4096
STEPS = 4
KT = 1024
NK = STEPS * KB
D = (512, 256, 256)
OFF = (0, 512, 768)

_IDX_BITS = (KT - 1).bit_length()
_IDX_MASK = (1 << _IDX_BITS) - 1
_ONE_BITS = 0x3F800000

_NC, _NS = 2, 16
_NW = _NC * _NS
_B_PER_W = BZ // _NW


def _search_body(query_ref, keys_ref, w_ref, vbits_ref, idx_ref, qp_ref,
                 kp0_ref, kp1_ref, kp2_ref, qnt_ref, wbf_ref):
    i = pl.program_id(0)
    j = pl.program_id(1)

    @pl.when((i == 0) & (j == 0))
    def _init():
        w = w_ref[...]
        wbf = w.astype(jnp.bfloat16)
        wbf_ref[...] = wbf
        qp = lax.dot(query_ref[...].astype(jnp.bfloat16), wbf,
                     preferred_element_type=jnp.float32)
        qp_ref[...] = qp
        for u in range(3):
            q = qp[:, OFF[u]:OFF[u] + D[u]]
            n = jnp.sqrt(jnp.sum(q * q, axis=1, keepdims=True))
            qn = q / jnp.maximum(n, 1e-8)
            qnt_ref[OFF[u]:OFF[u] + D[u], :] = qn.T.astype(jnp.bfloat16)
        vbits_ref[...] = jnp.full((8, BZ), _ONE_BITS, jnp.int32)
        idx_ref[...] = jnp.zeros((8, BZ), jnp.int32)

    keys_bf = keys_ref[0].astype(jnp.bfloat16)
    kp = lax.dot(keys_bf, wbf_ref[...],
                 preferred_element_type=jnp.float32)
    base = i * KB + j * KT
    row = lax.broadcasted_iota(jnp.int32, (KT, BZ), 0)
    rbits = _IDX_MASK - row
    for u, kpu_ref in ((0, kp0_ref), (1, kp1_ref), (2, kp2_ref)):
        kpu = kp[:, OFF[u]:OFF[u] + D[u]]
        kpu_ref[...] = kpu
        inv = 1.0 / jnp.maximum(
            jnp.sqrt(jnp.sum(kpu * kpu, axis=1, keepdims=True)), 1e-3)
        kpn = (kpu * inv).astype(jnp.bfloat16)
        st = lax.dot(kpn, qnt_ref[OFF[u]:OFF[u] + D[u], :],
                     preferred_element_type=jnp.float32)
        packed = (lax.bitcast_convert_type(st + 2.0, jnp.int32)
                  & jnp.int32(~_IDX_MASK)) | rbits
        pmax = jnp.max(packed, axis=0)
        pv = pmax & jnp.int32(~_IDX_MASK)
        gidx = base + (_IDX_MASK - (pmax & _IDX_MASK))
        better = pv > vbits_ref[u, :]
        vbits_ref[u, :] = jnp.where(better, pv, vbits_ref[u, :])
        idx_ref[u, :] = jnp.where(better, gidx, idx_ref[u, :])


_search = pl.pallas_call(
    _search_body,
    grid=(STEPS, KB // KT),
    in_specs=[
        pl.BlockSpec((BZ, H), lambda i, j: (0, 0)),
        pl.BlockSpec((1, KT, H), lambda i, j: (i, j, 0)),
        pl.BlockSpec((H, H), lambda i, j: (0, 0)),
    ],
    out_specs=[
        pl.BlockSpec((8, BZ), lambda i, j: (0, 0)),
        pl.BlockSpec((8, BZ), lambda i, j: (0, 0)),
        pl.BlockSpec((BZ, H), lambda i, j: (0, 0)),
        pl.BlockSpec((KT, D[0]), lambda i, j: (i * (KB // KT) + j, 0)),
        pl.BlockSpec((KT, D[1]), lambda i, j: (i * (KB // KT) + j, 0)),
        pl.BlockSpec((KT, D[2]), lambda i, j: (i * (KB // KT) + j, 0)),
    ],
    out_shape=[
        jax.ShapeDtypeStruct((8, BZ), jnp.int32),
        jax.ShapeDtypeStruct((8, BZ), jnp.int32),
        jax.ShapeDtypeStruct((BZ, H), jnp.float32),
        jax.ShapeDtypeStruct((NK, D[0]), jnp.float32),
        jax.ShapeDtypeStruct((NK, D[1]), jnp.float32),
        jax.ShapeDtypeStruct((NK, D[2]), jnp.float32),
    ],
    scratch_shapes=[
        pltpu.VMEM((H, BZ), jnp.bfloat16),
        pltpu.VMEM((H, H), jnp.bfloat16),
    ],
)


def _gather_diag(kp0, kp1, kp2, idx):
    mesh = plsc.VectorSubcoreMesh(core_axis_name="c", subcore_axis_name="s")

    @functools.partial(
        pl.kernel, mesh=mesh,
        out_type=[
            jax.ShapeDtypeStruct((BZ, D[0]), jnp.float32),
            jax.ShapeDtypeStruct((BZ, D[1]), jnp.float32),
            jax.ShapeDtypeStruct((BZ, D[2]), jnp.float32),
        ],
        scratch_types=[
            pltpu.VMEM((_B_PER_W,), jnp.int32),
            pltpu.VMEM((_B_PER_W, D[0]), jnp.float32),
            pltpu.VMEM((_B_PER_W, D[1]), jnp.float32),
            pltpu.VMEM((_B_PER_W, D[2]), jnp.float32),
            pltpu.SemaphoreType.DMA,
        ],
    )
    def k(kp0_hbm, kp1_hbm, kp2_hbm, idx_hbm,
          o0_hbm, o1_hbm, o2_hbm, idx_v, r0_v, r1_v, r2_v, sem):
        wid = lax.axis_index("s") * _NC + lax.axis_index("c")
        base = wid * _B_PER_W
        for u, kp_hbm, r_v, o_hbm in ((0, kp0_hbm, r0_v, o0_hbm),
                                      (1, kp1_hbm, r1_v, o1_hbm),
                                      (2, kp2_hbm, r2_v, o2_hbm)):
            pltpu.sync_copy(idx_hbm.at[u, pl.ds(base, _B_PER_W)], idx_v)
            pltpu.async_copy(kp_hbm.at[idx_v], r_v, sem).wait()
            pltpu.sync_copy(r_v, o_hbm.at[pl.ds(base, _B_PER_W)])

    return k(kp0, kp1, kp2, idx)


def _final_body(qp_ref, d0_ref, d1_ref, d2_ref, vbits_ref, out_ref):
    acc = jnp.float32(0.0)
    for u, d_ref in ((0, d0_ref), (1, d1_ref), (2, d2_ref)):
        diag = d_ref[...]
        alive = vbits_ref[u, :] > jnp.int32(_ONE_BITS)
        q = qp_ref[:, OFF[u]:OFF[u] + D[u]]
        dot = jnp.sum(q * diag, axis=1)
        nq = jnp.sqrt(jnp.sum(q * q, axis=1))
        nd = jnp.sqrt(jnp.sum(diag * diag, axis=1))
        cos = jnp.where(alive, dot / jnp.maximum(nq * nd, 1e-8), 0.0)
        acc = acc + D[u] * jnp.sum(cos)
    loss = -acc / (BZ * H)
    out_ref[...] = jnp.full((8, 128), loss, jnp.float32)


_final = pl.pallas_call(
    _final_body,
    in_specs=[
        pl.BlockSpec((BZ, H), lambda: (0, 0)),
        pl.BlockSpec((BZ, D[0]), lambda: (0, 0)),
        pl.BlockSpec((BZ, D[1]), lambda: (0, 0)),
        pl.BlockSpec((BZ, D[2]), lambda: (0, 0)),
        pl.BlockSpec((8, BZ), lambda: (0, 0)),
    ],
    out_specs=pl.BlockSpec((8, 128), lambda: (0, 0)),
    out_shape=jax.ShapeDtypeStruct((8, 128), jnp.float32),
)


def kernel(query, keys, W):
    vbits, idx, qp, kp0, kp1, kp2 = _search(query, keys, W)
    d0, d1, d2 = _gather_diag(kp0, kp1, kp2, idx)
    out = _final(qp, d0, d1, d2, vbits)
    return out[0, 0]
